# Initial kernel scaffold; baseline (speedup 1.0000x reference)
import jax
import jax.numpy as jnp
from jax.experimental import pallas as pl


def kernel(src, pos_coeff, src_subtoken_mask, tok_embedding, W_level, W_parent, lin_W, lin_b, ln_g, ln_b):
    raise NotImplementedError("write your pallas kernel here")


# speedup vs baseline: 9.6288x; 9.6288x over previous
import functools

import jax
import jax.numpy as jnp
from jax import lax
from jax.experimental import pallas as pl
from jax.experimental.pallas import tpu as pltpu
from jax.experimental.pallas import tpu_sc as plsc

_B, _L, _N, _S, _H = 16, 512, 16, 5, 64
_NT = _B * _L * _N
_NC, _NS = 2, 16
_NW = _NC * _NS
_NPW = _NT // _NW
_CH = 64
_NCHUNK = _NPW // _CH
_ROWS = _NT // _N
_F = _N * _H


def _sc_body(table, idx3, mask3, out2d, idx_v, mask_v, rows_v, out_v,
             g0, g1, o0, o1):
    gsem = (g0, g1)
    osem = (o0, o1)
    wid = lax.axis_index("s") * _NC + lax.axis_index("c")
    row0 = wid * _NCHUNK
    pltpu.sync_copy(idx3.at[:, pl.ds(row0, _NCHUNK)], idx_v)
    pltpu.sync_copy(mask3.at[:, pl.ds(row0, _NCHUNK)], mask_v)

    def fire(j, b):
        for s in range(_S):
            pltpu.async_copy(table.at[idx_v.at[s, j]], rows_v.at[b, s],
                             gsem[b])

    def gwait(b):
        for s in range(_S):
            pltpu.make_async_copy(table.at[idx_v.at[s, 0]],
                                  rows_v.at[b, s], gsem[b]).wait()

    def owait(b):
        pltpu.make_async_copy(out_v.at[b], out2d.at[pl.ds(0, _CH // _N)],
                              osem[b]).wait()

    def compute(j, b):
        def group(g, carry):
            base = g * 16
            mvs = [mask_v[s, j, pl.ds(base, 16)] for s in range(_S)]
            for ii in range(16):
                ms = [mvs[s][ii] for s in range(_S)]
                for k in range(_H // 16):
                    sl = pl.ds(k * 16, 16)
                    acc = ms[0] * rows_v[b, 0, base + ii, sl]
                    for s in range(1, _S):
                        acc = acc + ms[s] * rows_v[b, s, base + ii, sl]
                    out_v[b, g, pl.ds(ii * _H + k * 16, 16)] = acc
            return carry
        lax.fori_loop(0, _CH // 16, group, 0)

    fire(0, 0)
    fire(1, 1)

    def step(t, carry):
        for b in range(2):
            j = 2 * t + b
            gwait(b)

            @pl.when(j >= 2)
            def _():
                owait(b)

            compute(j, b)
            orow = wid * (_NPW // _N) + j * (_CH // _N)
            pltpu.async_copy(out_v.at[b], out2d.at[pl.ds(orow, _CH // _N)],
                             osem[b])

            @pl.when(j + 2 < _NCHUNK)
            def _():
                fire(j + 2, b)
        return carry

    lax.fori_loop(0, _NCHUNK // 2, step, 0)
    owait(0)
    owait(1)


@functools.cache
def _sc_gather_fn():
    return pl.kernel(
        _sc_body,
        out_type=jax.ShapeDtypeStruct((_ROWS, _F), jnp.float32),
        mesh=plsc.VectorSubcoreMesh(core_axis_name="c", subcore_axis_name="s",
                                    num_cores=_NC, num_subcores=_NS),
        scratch_types=[
            pltpu.VMEM((_S, _NCHUNK, _CH), jnp.int32),
            pltpu.VMEM((_S, _NCHUNK, _CH), jnp.float32),
            pltpu.VMEM((2, _S, _CH, _H), jnp.float32),
            pltpu.VMEM((2, _CH // _N, _F), jnp.float32),
            pltpu.SemaphoreType.DMA,
            pltpu.SemaphoreType.DMA,
            pltpu.SemaphoreType.DMA,
            pltpu.SemaphoreType.DMA,
        ],
        compiler_params=pltpu.CompilerParams(use_tc_tiling_on_sc=False),
    )


def _tc_body(x_ref, pc_ref, wl_ref, wp_ref, lw_ref, lb_ref, g_ref, bb_ref,
             o_ref):
    hp = lax.Precision.HIGHEST
    x = x_ref[...]
    pc = pc_ref[...]
    lw = lw_ref[...]
    wl = wl_ref[...]
    d = wp_ref[...] - wl
    f_id = lax.broadcasted_iota(jnp.int32, (_F, _N), 0)
    n_id = lax.broadcasted_iota(jnp.int32, (_F, _N), 1)
    sel = jnp.where(f_id // _H == n_id, 1.0, 0.0)
    a_t = lax.dot_general(lw * d, sel, (((1,), (0,)), ((), ())),
                          precision=hp, preferred_element_type=jnp.float32)
    c = lax.dot_general(wl, lw, (((1,), (1,)), ((), ())),
                        precision=hp, preferred_element_type=jnp.float32)
    y = lax.dot_general(x, lw, (((1,), (1,)), ((), ())),
                        precision=hp, preferred_element_type=jnp.float32)
    y = y + lax.dot_general(pc, a_t, (((1,), (1,)), ((), ())),
                            precision=hp, preferred_element_type=jnp.float32)
    y = y + c + lb_ref[...]
    mu = jnp.mean(y, axis=1, keepdims=True)
    yc = y - mu
    var = jnp.mean(yc * yc, axis=1, keepdims=True)
    o_ref[...] = yc * lax.rsqrt(var + 1e-5) * g_ref[...] + bb_ref[...]


_BR = 512

_tc_mix = pl.pallas_call(
    _tc_body,
    out_shape=jax.ShapeDtypeStruct((_ROWS, _H), jnp.float32),
    grid=(_ROWS // _BR,),
    in_specs=[
        pl.BlockSpec((_BR, _F), lambda i: (i, 0)),
        pl.BlockSpec((_BR, _N), lambda i: (i, 0)),
        pl.BlockSpec((1, _F), lambda i: (0, 0)),
        pl.BlockSpec((1, _F), lambda i: (0, 0)),
        pl.BlockSpec((_H, _F), lambda i: (0, 0)),
        pl.BlockSpec((1, _H), lambda i: (0, 0)),
        pl.BlockSpec((1, _H), lambda i: (0, 0)),
        pl.BlockSpec((1, _H), lambda i: (0, 0)),
    ],
    out_specs=pl.BlockSpec((_BR, _H), lambda i: (i, 0)),
)


def kernel(src, pos_coeff, src_subtoken_mask, tok_embedding, W_level,
           W_parent, lin_W, lin_b, ln_g, ln_b):
    idx3 = src.reshape(_NT, _S).T.reshape(_S, _NT // _CH, _CH)
    mask3 = src_subtoken_mask.reshape(_NT, _S).T.reshape(_S, _NT // _CH, _CH)
    x2 = _sc_gather_fn()(tok_embedding, idx3, mask3)
    out = _tc_mix(x2, pos_coeff.reshape(_ROWS, _N),
                  W_level.reshape(1, _F), W_parent.reshape(1, _F), lin_W,
                  lin_b.reshape(1, _H), ln_g.reshape(1, _H),
                  ln_b.reshape(1, _H))
    return out.reshape(_B, _L, _H)
